# baseline (speedup 1.0000x reference)
import functools

import jax
import jax.numpy as jnp
from jax import lax
from jax.experimental import pallas as pl
from jax.experimental.pallas import tpu as pltpu
from jax.experimental.pallas import tpu_sc as plsc

N = 10000
E = 320000
DH = 128
DE = 16
NLAYERS = 4

NC, NS = 2, 16
NW = NC * NS
CB = 128
CH = 80
EP = NW * CH * CB
NCHUNK = NW * CH
NACC = 10240
PFLAT = NACC * 3
NPACK = PFLAT // DH
NROWS_TILE = NACC // NS

EB = 512
CPB = EB // CB
NB = 1024

_f32 = jnp.float32
_i32 = jnp.int32


@functools.cache
def _build_sc_kernels():
    mesh = plsc.VectorSubcoreMesh(core_axis_name="c", subcore_axis_name="s")

    @functools.partial(
        pl.kernel,
        out_type=[
            jax.ShapeDtypeStruct((EP, DH), _f32),
            jax.ShapeDtypeStruct((EP, DH), _f32),
            jax.ShapeDtypeStruct((NCHUNK, 4, CB), _f32),
        ],
        mesh=mesh,
        scratch_types=[
            pltpu.VMEM((CH, CB), _i32),
            pltpu.VMEM((CH, CB), _i32),
            pltpu.VMEM((PFLAT,), _f32),
            pltpu.VMEM((CB, DH), _f32),
            pltpu.VMEM((CB, DH), _f32),
            pltpu.VMEM((4, CB), _f32),
            pltpu.VMEM((CB, DH), _f32),
            pltpu.VMEM((CB, DH), _f32),
            pltpu.VMEM((4, CB), _f32),
            pltpu.SemaphoreType.DMA,
            pltpu.SemaphoreType.DMA,
            pltpu.SemaphoreType.DMA,
        ],
        compiler_params=pltpu.CompilerParams(needs_layout_passes=False),
    )
    def gather_sc(t_hbm, posp_hbm, rowg_hbm, colg_hbm, src_out, dst_out,
                  aux_out, idxr_v, idxc_v, posp_v, bufr0, bufc0, aux0,
                  bufr1, bufc1, aux1, gsem, wsem0, wsem1):
        wid = lax.axis_index("s") * NC + lax.axis_index("c")
        pltpu.sync_copy(rowg_hbm.at[wid], idxr_v)
        pltpu.sync_copy(colg_hbm.at[wid], idxc_v)
        pltpu.sync_copy(posp_hbm, posp_v)
        base = wid * (CH * CB)
        cbase = wid * CH

        def gather(j, br, bc):
            cr = pltpu.async_copy(t_hbm.at[idxr_v.at[j]], br, gsem)
            cc = pltpu.async_copy(t_hbm.at[idxc_v.at[j]], bc, gsem)
            cr.wait()
            cc.wait()

        def aux_compute(j, av):
            for g in range(CB // 16):
                sl = pl.ds(g * 16, 16)
                ar = idxr_v[j, sl] * 3
                ac = idxc_v[j, sl] * 3
                prx = plsc.load_gather(posp_v, [ar])
                pry = plsc.load_gather(posp_v, [ar + 1])
                prz = plsc.load_gather(posp_v, [ar + 2])
                pcx = plsc.load_gather(posp_v, [ac])
                pcy = plsc.load_gather(posp_v, [ac + 1])
                pcz = plsc.load_gather(posp_v, [ac + 2])
                cdx = prx - pcx
                cdy = pry - pcy
                cdz = prz - pcz
                av[0, sl] = cdx
                av[1, sl] = cdy
                av[2, sl] = cdz
                av[3, sl] = cdx * cdx + cdy * cdy + cdz * cdz

        def writes(j, br, bc, av, ws):
            off = base + j * CB
            pltpu.async_copy(br, src_out.at[pl.ds(off, CB)], ws)
            pltpu.async_copy(bc, dst_out.at[pl.ds(off, CB)], ws)
            pltpu.async_copy(av, aux_out.at[cbase + j], ws)

        def drain_writes(j, br, bc, av, ws):
            off = base + j * CB
            pltpu.make_async_copy(br, src_out.at[pl.ds(off, CB)], ws).wait()
            pltpu.make_async_copy(bc, dst_out.at[pl.ds(off, CB)], ws).wait()
            pltpu.make_async_copy(av, aux_out.at[cbase + j], ws).wait()

        gather(0, bufr0, bufc0)
        aux_compute(0, aux0)
        writes(0, bufr0, bufc0, aux0, wsem0)
        gather(1, bufr1, bufc1)
        aux_compute(1, aux1)
        writes(1, bufr1, bufc1, aux1, wsem1)

        def body(it, carry):
            j0 = 2 * it
            drain_writes(j0, bufr0, bufc0, aux0, wsem0)
            gather(j0, bufr0, bufc0)
            aux_compute(j0, aux0)
            writes(j0, bufr0, bufc0, aux0, wsem0)
            j1 = j0 + 1
            drain_writes(j1, bufr1, bufc1, aux1, wsem1)
            gather(j1, bufr1, bufc1)
            aux_compute(j1, aux1)
            writes(j1, bufr1, bufc1, aux1, wsem1)
            return carry

        lax.fori_loop(1, CH // 2, body, 0)
        drain_writes(CH - 2, bufr0, bufc0, aux0, wsem0)
        drain_writes(CH - 1, bufr1, bufc1, aux1, wsem1)

    @functools.partial(
        pl.kernel,
        out_type=jax.ShapeDtypeStruct((NC, NACC, DH), _f32),
        mesh=mesh,
        scratch_types=[
            pltpu.VMEM((1, CB), _i32),
            pltpu.VMEM((CB, DH), _f32),
            pltpu.VMEM((1, CB), _i32),
            pltpu.VMEM((CB, DH), _f32),
            pltpu.VMEM_SHARED((NACC, DH), _f32),
            pltpu.SemaphoreType.DMA,
            pltpu.SemaphoreType.DMA,
        ],
        compiler_params=pltpu.CompilerParams(needs_layout_passes=False),
    )
    def scatter_ef_sc(payf_hbm, cols_hbm, zeros_hbm, outf_hbm,
                      idx0, bufe0, idx1, bufe1, accf_sh, asem0, asem1):
        cid = lax.axis_index("c")
        sid = lax.axis_index("s")
        wid = sid * NC + cid
        tsl = pl.ds(sid * NROWS_TILE, NROWS_TILE)
        pltpu.sync_copy(zeros_hbm, accf_sh.at[tsl])
        plsc.subcore_barrier()
        base = wid * (CH * CB)

        def load(j, iv, bv):
            pltpu.sync_copy(cols_hbm.at[wid, pl.ds(j, 1)], iv)
            pltpu.sync_copy(payf_hbm.at[pl.ds(base + j * CB, CB)], bv)

        def add(iv, bv, sem):
            pltpu.async_copy(bv, accf_sh.at[iv.at[0]], sem, add=True)

        def drain_add(iv, bv, sem):
            pltpu.make_async_copy(bv, accf_sh.at[iv.at[0]], sem).wait()

        load(0, idx0, bufe0)
        add(idx0, bufe0, asem0)
        load(1, idx1, bufe1)
        add(idx1, bufe1, asem1)

        def body(it, carry):
            j0 = 2 * it
            drain_add(idx0, bufe0, asem0)
            load(j0, idx0, bufe0)
            add(idx0, bufe0, asem0)
            drain_add(idx1, bufe1, asem1)
            load(j0 + 1, idx1, bufe1)
            add(idx1, bufe1, asem1)
            return carry

        lax.fori_loop(1, CH // 2, body, 0)
        drain_add(idx0, bufe0, asem0)
        drain_add(idx1, bufe1, asem1)
        plsc.subcore_barrier()
        pltpu.sync_copy(accf_sh.at[tsl], outf_hbm.at[cid, tsl])

    @functools.partial(
        pl.kernel,
        out_type=jax.ShapeDtypeStruct((NW, PFLAT), _f32),
        mesh=mesh,
        scratch_types=[
            pltpu.VMEM((CH, CB), _i32),
            pltpu.VMEM((4, CB), _f32),
            pltpu.VMEM((PFLAT,), _f32),
        ],
        compiler_params=pltpu.CompilerParams(needs_layout_passes=False),
    )
    def scatter_coord_sc(payc_hbm, cols_hbm, zflat_hbm, outc_hbm,
                         idx_v, bufc_v, accc_v):
        cid = lax.axis_index("c")
        sid = lax.axis_index("s")
        wid = sid * NC + cid
        pltpu.sync_copy(zflat_hbm, accc_v)
        pltpu.sync_copy(cols_hbm.at[wid], idx_v)

        def body(j, carry):
            pltpu.sync_copy(payc_hbm.at[wid * CH + j], bufc_v)
            for g in range(CB // 16):
                sl = pl.ds(g * 16, 16)
                addr = idx_v[j, sl] * 3
                for c in range(3):
                    plsc.addupdate_scatter(accc_v, [addr + c], bufc_v[c, sl])
            return carry

        lax.fori_loop(0, CH, body, 0)
        pltpu.sync_copy(accc_v, outc_hbm.at[wid])

    return gather_sc, scatter_ef_sc, scatter_coord_sc


def _edge_body(src_ref, dst_ref, aux_ref, ea_ref, w1a, w1b, w1d, w1e, b1,
               w2, b2, cw1, cb1, cw2r, outf_ref, outc_ref):
    xj = src_ref[...]
    xi = dst_ref[...]
    s0 = lax.broadcasted_iota(_i32, (EB, CPB), 0)
    r0 = lax.broadcasted_iota(_i32, (EB, CPB), 1)
    rmat = (lax.shift_right_logical(s0, 7) == r0).astype(_f32)
    s1 = lax.broadcasted_iota(_i32, (EB, CB), 0)
    e1 = lax.broadcasted_iota(_i32, (EB, CB), 1)
    mmat = (jnp.bitwise_and(s1, CB - 1) == e1).astype(_f32)

    def pick(c):
        a = jnp.dot(rmat, aux_ref[:, c, :], preferred_element_type=_f32)
        return jnp.sum(a * mmat, axis=1, keepdims=True)

    cdx, cdy, cdz, dist = pick(0), pick(1), pick(2), pick(3)
    z = (jnp.dot(xi, w1a[...], preferred_element_type=_f32)
         + jnp.dot(xj, w1b[...], preferred_element_type=_f32)
         + dist * w1d[...]
         + jnp.dot(ea_ref[...], w1e[...], preferred_element_type=_f32)
         + b1[...])
    e1_ = z * jax.nn.sigmoid(z)
    z2 = jnp.dot(e1_, w2[...], preferred_element_type=_f32) + b2[...]
    ef = z2 * jax.nn.sigmoid(z2)
    z3 = jnp.dot(ef, cw1[...], preferred_element_type=_f32) + cb1[...]
    c1 = z3 * jax.nn.sigmoid(z3)
    s = jnp.sum(c1 * cw2r[...], axis=1, keepdims=True)
    outf_ref[...] = ef
    for c, cd in ((0, cdx), (1, cdy), (2, cdz)):
        w = (cd * s) * mmat
        outc_ref[:, c, :] = lax.dot_general(
            rmat, w, (((0,), (0,)), ((), ())), preferred_element_type=_f32)
    outc_ref[:, 3, :] = jnp.zeros((CPB, CB), _f32)


def _full2d(shape):
    return pl.BlockSpec(shape, lambda i: (0, 0))


_edge_call = pl.pallas_call(
    _edge_body,
    grid=(EP // EB,),
    in_specs=[
        pl.BlockSpec((EB, DH), lambda i: (i, 0)),
        pl.BlockSpec((EB, DH), lambda i: (i, 0)),
        pl.BlockSpec((CPB, 4, CB), lambda i: (i, 0, 0)),
        pl.BlockSpec((EB, DE), lambda i: (i, 0)),
        _full2d((DH, DH)), _full2d((DH, DH)), _full2d((1, DH)),
        _full2d((DE, DH)), _full2d((1, DH)),
        _full2d((DH, DH)), _full2d((1, DH)),
        _full2d((DH, DH)), _full2d((1, DH)), _full2d((1, DH)),
    ],
    out_specs=[
        pl.BlockSpec((EB, DH), lambda i: (i, 0)),
        pl.BlockSpec((CPB, 4, CB), lambda i: (i, 0, 0)),
    ],
    out_shape=[
        jax.ShapeDtypeStruct((EP, DH), _f32),
        jax.ShapeDtypeStruct((NCHUNK, 4, CB), _f32),
    ],
)


def _node_common(t_ref, pp_ref, pf_ref, pc_ref, nw1a, nw1b, nb1, nw2, nb2):
    h = t_ref[...]
    pf = pf_ref[...]
    aggf = pf[0] + pf[1]
    z = (jnp.dot(h, nw1a[...], preferred_element_type=_f32)
         + jnp.dot(aggf, nw1b[...], preferred_element_type=_f32)
         + nb1[...])
    u = z * jax.nn.sigmoid(z)
    h_new = jnp.dot(u, nw2[...], preferred_element_type=_f32) + nb2[...]
    pp_new = pp_ref[...] + jnp.sum(pc_ref[...], axis=0)
    return h_new, pp_new


def _node_body(t_ref, pp_ref, pf_ref, pc_ref, nw1a, nw1b, nb1, nw2, nb2,
               h_out_ref, pp_out_ref):
    h_new, pp_new = _node_common(t_ref, pp_ref, pf_ref, pc_ref,
                                 nw1a, nw1b, nb1, nw2, nb2)
    h_out_ref[...] = h_new
    pp_out_ref[...] = pp_new


def _node_final_body(t_ref, pp_ref, pf_ref, pc_ref, nw1a, nw1b, nb1, nw2, nb2,
                     wout, bout, x_out_ref, pp_out_ref):
    h_new, pp_new = _node_common(t_ref, pp_ref, pf_ref, pc_ref,
                                 nw1a, nw1b, nb1, nw2, nb2)
    x_out_ref[...] = (jnp.dot(h_new, wout[...],
                              preferred_element_type=_f32) + bout[...])
    pp_out_ref[...] = pp_new


_PPB = NB * 3 // DH

_node_in_specs = [
    pl.BlockSpec((NB, DH), lambda i: (i, 0)),
    pl.BlockSpec((_PPB, DH), lambda i: (i, 0)),
    pl.BlockSpec((NC, NB, DH), lambda i: (0, i, 0)),
    pl.BlockSpec((NW, _PPB, DH), lambda i: (0, i, 0)),
    _full2d((DH, DH)), _full2d((DH, DH)), _full2d((1, DH)),
    _full2d((DH, DH)), _full2d((1, DH)),
]
_node_out_specs = [
    pl.BlockSpec((NB, DH), lambda i: (i, 0)),
    pl.BlockSpec((_PPB, DH), lambda i: (i, 0)),
]
_node_out_shape = [
    jax.ShapeDtypeStruct((N, DH), _f32),
    jax.ShapeDtypeStruct((NPACK, DH), _f32),
]

_node_call = pl.pallas_call(
    _node_body, grid=(NACC // NB,), in_specs=_node_in_specs,
    out_specs=_node_out_specs, out_shape=_node_out_shape)

_node_final_call = pl.pallas_call(
    _node_final_body, grid=(NACC // NB,),
    in_specs=_node_in_specs + [_full2d((DH, DH)), _full2d((1, DH))],
    out_specs=_node_out_specs, out_shape=_node_out_shape)


def _init_body(x_ref, w_ref, b_ref, out_ref):
    out_ref[...] = jnp.dot(x_ref[...], w_ref[...],
                           preferred_element_type=_f32) + b_ref[...]


_init_call = pl.pallas_call(
    _init_body,
    grid=(NACC // NB,),
    in_specs=[
        pl.BlockSpec((NB, DH), lambda i: (i, 0)),
        _full2d((DH, DH)), _full2d((1, DH)),
    ],
    out_specs=pl.BlockSpec((NB, DH), lambda i: (i, 0)),
    out_shape=jax.ShapeDtypeStruct((N, DH), _f32),
)


def kernel(x, pos, edge_index, edge_attr, W_in, b_in, edge_W1, edge_b1,
           edge_W2, edge_b2, coord_W1, coord_b1, coord_W2, node_W1, node_b1,
           node_W2, node_b2, W_out, b_out):
    row = edge_index[0]
    col = edge_index[1]
    padn = EP - E
    rowg = jnp.concatenate([row, jnp.zeros((padn,), _i32)]).reshape(NW, CH, CB)
    colg = jnp.concatenate([col, jnp.zeros((padn,), _i32)]).reshape(NW, CH, CB)
    cols = jnp.concatenate([col, jnp.full((padn,), N, _i32)]).reshape(NW, CH, CB)
    ea_pad = jnp.concatenate([edge_attr, jnp.zeros((padn, DE), _f32)], axis=0)
    zeros_tile = jnp.zeros((NROWS_TILE, DH), _f32)
    zeros_flat = jnp.zeros((PFLAT,), _f32)
    posp = jnp.zeros((NACC, 3), _f32).at[:N].set(pos).reshape(NPACK, DH)

    gather_sc, scatter_ef_sc, scatter_coord_sc = _build_sc_kernels()
    b1r = lambda b: b.reshape(1, DH)
    t = _init_call(x, W_in, b_in.reshape(1, DH))
    for l in range(NLAYERS):
        src, dst, aux = gather_sc(t, posp.reshape(PFLAT), rowg, colg)
        payf, payc = _edge_call(
            src, dst, aux, ea_pad,
            edge_W1[l, :DH], edge_W1[l, DH:2 * DH],
            edge_W1[l, 2 * DH:2 * DH + 1], edge_W1[l, 2 * DH + 1:],
            b1r(edge_b1[l]),
            edge_W2[l], b1r(edge_b2[l]),
            coord_W1[l], b1r(coord_b1[l]),
            coord_W2[l].reshape(1, DH),
        )
        pf = scatter_ef_sc(payf, cols, zeros_tile)
        pc = scatter_coord_sc(payc, cols, zeros_flat)
        args = (t, posp, pf, pc.reshape(NW, NPACK, DH),
                node_W1[l, :DH], node_W1[l, DH:], b1r(node_b1[l]),
                node_W2[l], b1r(node_b2[l]))
        if l < NLAYERS - 1:
            t, posp = _node_call(*args)
        else:
            out_x, posp = _node_final_call(*args, W_out, b_out.reshape(1, DH))
    pos_out = posp.reshape(NACC, 3)[:N]
    return (out_x, pos_out)


# speedup vs baseline: 1.1219x; 1.1219x over previous
import functools

import jax
import jax.numpy as jnp
from jax import lax
from jax.experimental import pallas as pl
from jax.experimental.pallas import tpu as pltpu
from jax.experimental.pallas import tpu_sc as plsc

N = 10000
E = 320000
DH = 128
DE = 16
NLAYERS = 4

NC, NS = 2, 16
NW = NC * NS
CB = 128
CH = 80
EP = NW * CH * CB
NCHUNK = NW * CH
NACC = 10240
PFLAT = NACC * 3
NPACK = PFLAT // DH
NROWS_TILE = NACC // NS

EB = 512
CPB = EB // CB
NB = 1024

_f32 = jnp.float32
_i32 = jnp.int32


@functools.cache
def _build_sc_kernels():
    mesh = plsc.VectorSubcoreMesh(core_axis_name="c", subcore_axis_name="s")

    @functools.partial(
        pl.kernel,
        out_type=[
            jax.ShapeDtypeStruct((EP, DH), _f32),
            jax.ShapeDtypeStruct((EP, DH), _f32),
            jax.ShapeDtypeStruct((NCHUNK, 4, CB), _f32),
        ],
        mesh=mesh,
        scratch_types=[
            pltpu.VMEM((CH, CB), _i32),
            pltpu.VMEM((CH, CB), _i32),
            pltpu.VMEM((PFLAT,), _f32),
            pltpu.VMEM((CB, DH), _f32),
            pltpu.VMEM((CB, DH), _f32),
            pltpu.VMEM((4, CB), _f32),
            pltpu.SemaphoreType.DMA,
            pltpu.SemaphoreType.DMA,
        ],
        compiler_params=pltpu.CompilerParams(needs_layout_passes=False),
    )
    def gather_sc(pb_hbm, pa_hbm, posp_hbm, rowg_hbm, colg_hbm, src_out,
                  dst_out, aux_out, idxr_v, idxc_v, posp_v, bufr_v, bufc_v,
                  aux_v, semr, semc):
        wid = lax.axis_index("s") * NC + lax.axis_index("c")
        pltpu.sync_copy(rowg_hbm.at[wid], idxr_v)
        pltpu.sync_copy(colg_hbm.at[wid], idxc_v)
        pltpu.sync_copy(posp_hbm, posp_v)
        base = wid * (CH * CB)
        cbase = wid * CH

        def body(j, carry):
            off = base + j * CB
            cr = pltpu.async_copy(pb_hbm.at[idxr_v.at[j]], bufr_v, semr)
            cc = pltpu.async_copy(pa_hbm.at[idxc_v.at[j]], bufc_v, semc)
            for g in range(CB // 16):
                sl = pl.ds(g * 16, 16)
                ar = idxr_v[j, sl] * 3
                ac = idxc_v[j, sl] * 3
                prx = plsc.load_gather(posp_v, [ar])
                pry = plsc.load_gather(posp_v, [ar + 1])
                prz = plsc.load_gather(posp_v, [ar + 2])
                pcx = plsc.load_gather(posp_v, [ac])
                pcy = plsc.load_gather(posp_v, [ac + 1])
                pcz = plsc.load_gather(posp_v, [ac + 2])
                cdx = prx - pcx
                cdy = pry - pcy
                cdz = prz - pcz
                aux_v[0, sl] = cdx
                aux_v[1, sl] = cdy
                aux_v[2, sl] = cdz
                aux_v[3, sl] = cdx * cdx + cdy * cdy + cdz * cdz
            cr.wait()
            pltpu.sync_copy(bufr_v, src_out.at[pl.ds(off, CB)])
            cc.wait()
            pltpu.sync_copy(bufc_v, dst_out.at[pl.ds(off, CB)])
            pltpu.sync_copy(aux_v, aux_out.at[cbase + j])
            return carry

        lax.fori_loop(0, CH, body, 0)

    @functools.partial(
        pl.kernel,
        out_type=[
            jax.ShapeDtypeStruct((NC, NACC, DH), _f32),
            jax.ShapeDtypeStruct((NW, PFLAT), _f32),
        ],
        mesh=mesh,
        scratch_types=[
            pltpu.VMEM((1, CB), _i32),
            pltpu.VMEM((CB, DH), _f32),
            pltpu.VMEM((4, CB), _f32),
            pltpu.VMEM((PFLAT,), _f32),
            pltpu.VMEM_SHARED((NACC, DH), _f32),
        ],
        compiler_params=pltpu.CompilerParams(needs_layout_passes=False),
    )
    def scatter_sc(payf_hbm, payc_hbm, cols_hbm, zeros_hbm, zflat_hbm,
                   outf_hbm, outc_hbm, idx_v, bufe_v, bufc_v, accc_v, accf_sh):
        cid = lax.axis_index("c")
        sid = lax.axis_index("s")
        wid = sid * NC + cid
        tsl = pl.ds(sid * NROWS_TILE, NROWS_TILE)
        pltpu.sync_copy(zeros_hbm, accf_sh.at[tsl])
        pltpu.sync_copy(zflat_hbm, accc_v)
        plsc.subcore_barrier()
        base = wid * (CH * CB)

        def body(j, carry):
            off = base + j * CB
            pltpu.sync_copy(cols_hbm.at[wid, pl.ds(j, 1)], idx_v)
            pltpu.sync_copy(payf_hbm.at[pl.ds(off, CB)], bufe_v)
            pltpu.sync_copy(payc_hbm.at[wid * CH + j], bufc_v)
            pltpu.sync_copy(bufe_v, accf_sh.at[idx_v.at[0]], add=True)
            for g in range(CB // 16):
                sl = pl.ds(g * 16, 16)
                addr = idx_v[0, sl] * 3
                for c in range(3):
                    plsc.addupdate_scatter(accc_v, [addr + c], bufc_v[c, sl])
            return carry

        lax.fori_loop(0, CH, body, 0)
        pltpu.sync_copy(accc_v, outc_hbm.at[wid])
        plsc.subcore_barrier()
        pltpu.sync_copy(accf_sh.at[tsl], outf_hbm.at[cid, tsl])

    return gather_sc, scatter_sc


def _edge_body(src_ref, dst_ref, aux_ref, ea_ref, w1d, w1e, b1,
               w2, b2, cw1, cb1, cw2r, outf_ref, outc_ref):
    s0 = lax.broadcasted_iota(_i32, (EB, CPB), 0)
    r0 = lax.broadcasted_iota(_i32, (EB, CPB), 1)
    rmat = (lax.shift_right_logical(s0, 7) == r0).astype(_f32)
    s1 = lax.broadcasted_iota(_i32, (EB, CB), 0)
    e1 = lax.broadcasted_iota(_i32, (EB, CB), 1)
    mmat = (jnp.bitwise_and(s1, CB - 1) == e1).astype(_f32)

    def pick(c):
        a = jnp.dot(rmat, aux_ref[:, c, :], preferred_element_type=_f32)
        return jnp.sum(a * mmat, axis=1, keepdims=True)

    cdx, cdy, cdz, dist = pick(0), pick(1), pick(2), pick(3)
    z = (src_ref[...] + dst_ref[...]
         + dist * w1d[...]
         + jnp.dot(ea_ref[...], w1e[...], preferred_element_type=_f32)
         + b1[...])
    e1_ = z * jax.nn.sigmoid(z)
    z2 = jnp.dot(e1_, w2[...], preferred_element_type=_f32) + b2[...]
    ef = z2 * jax.nn.sigmoid(z2)
    z3 = jnp.dot(ef, cw1[...], preferred_element_type=_f32) + cb1[...]
    c1 = z3 * jax.nn.sigmoid(z3)
    s = jnp.sum(c1 * cw2r[...], axis=1, keepdims=True)
    outf_ref[...] = ef
    for c, cd in ((0, cdx), (1, cdy), (2, cdz)):
        w = (cd * s) * mmat
        outc_ref[:, c, :] = lax.dot_general(
            rmat, w, (((0,), (0,)), ((), ())), preferred_element_type=_f32)
    outc_ref[:, 3, :] = jnp.zeros((CPB, CB), _f32)


def _full2d(shape):
    return pl.BlockSpec(shape, lambda i: (0, 0))


_edge_call = pl.pallas_call(
    _edge_body,
    grid=(EP // EB,),
    in_specs=[
        pl.BlockSpec((EB, DH), lambda i: (i, 0)),
        pl.BlockSpec((EB, DH), lambda i: (i, 0)),
        pl.BlockSpec((CPB, 4, CB), lambda i: (i, 0, 0)),
        pl.BlockSpec((EB, DE), lambda i: (i, 0)),
        _full2d((1, DH)),
        _full2d((DE, DH)), _full2d((1, DH)),
        _full2d((DH, DH)), _full2d((1, DH)),
        _full2d((DH, DH)), _full2d((1, DH)), _full2d((1, DH)),
    ],
    out_specs=[
        pl.BlockSpec((EB, DH), lambda i: (i, 0)),
        pl.BlockSpec((CPB, 4, CB), lambda i: (i, 0, 0)),
    ],
    out_shape=[
        jax.ShapeDtypeStruct((EP, DH), _f32),
        jax.ShapeDtypeStruct((NCHUNK, 4, CB), _f32),
    ],
)


def _node_common(t_ref, pp_ref, pf_ref, pc_ref, nw1a, nw1b, nb1, nw2, nb2):
    h = t_ref[...]
    pf = pf_ref[...]
    aggf = pf[0] + pf[1]
    z = (jnp.dot(h, nw1a[...], preferred_element_type=_f32)
         + jnp.dot(aggf, nw1b[...], preferred_element_type=_f32)
         + nb1[...])
    u = z * jax.nn.sigmoid(z)
    h_new = jnp.dot(u, nw2[...], preferred_element_type=_f32) + nb2[...]
    pp_new = pp_ref[...] + jnp.sum(pc_ref[...], axis=0)
    return h_new, pp_new


def _node_body(t_ref, pp_ref, pf_ref, pc_ref, nw1a, nw1b, nb1, nw2, nb2,
               w1a_n, w1b_n, h_out_ref, pp_out_ref, pa_out_ref, pb_out_ref):
    h_new, pp_new = _node_common(t_ref, pp_ref, pf_ref, pc_ref,
                                 nw1a, nw1b, nb1, nw2, nb2)
    h_out_ref[...] = h_new
    pp_out_ref[...] = pp_new
    pa_out_ref[...] = jnp.dot(h_new, w1a_n[...], preferred_element_type=_f32)
    pb_out_ref[...] = jnp.dot(h_new, w1b_n[...], preferred_element_type=_f32)


def _node_final_body(t_ref, pp_ref, pf_ref, pc_ref, nw1a, nw1b, nb1, nw2, nb2,
                     wout, bout, x_out_ref, pp_out_ref):
    h_new, pp_new = _node_common(t_ref, pp_ref, pf_ref, pc_ref,
                                 nw1a, nw1b, nb1, nw2, nb2)
    x_out_ref[...] = (jnp.dot(h_new, wout[...],
                              preferred_element_type=_f32) + bout[...])
    pp_out_ref[...] = pp_new


_PPB = NB * 3 // DH

_node_in_specs = [
    pl.BlockSpec((NB, DH), lambda i: (i, 0)),
    pl.BlockSpec((_PPB, DH), lambda i: (i, 0)),
    pl.BlockSpec((NC, NB, DH), lambda i: (0, i, 0)),
    pl.BlockSpec((NW, _PPB, DH), lambda i: (0, i, 0)),
    _full2d((DH, DH)), _full2d((DH, DH)), _full2d((1, DH)),
    _full2d((DH, DH)), _full2d((1, DH)),
]
_node_out_specs = [
    pl.BlockSpec((NB, DH), lambda i: (i, 0)),
    pl.BlockSpec((_PPB, DH), lambda i: (i, 0)),
]
_node_out_shape = [
    jax.ShapeDtypeStruct((N, DH), _f32),
    jax.ShapeDtypeStruct((NPACK, DH), _f32),
]

_node_call = pl.pallas_call(
    _node_body, grid=(NACC // NB,),
    in_specs=_node_in_specs + [_full2d((DH, DH)), _full2d((DH, DH))],
    out_specs=_node_out_specs + [
        pl.BlockSpec((NB, DH), lambda i: (i, 0)),
        pl.BlockSpec((NB, DH), lambda i: (i, 0)),
    ],
    out_shape=_node_out_shape + [
        jax.ShapeDtypeStruct((N, DH), _f32),
        jax.ShapeDtypeStruct((N, DH), _f32),
    ])

_node_final_call = pl.pallas_call(
    _node_final_body, grid=(NACC // NB,),
    in_specs=_node_in_specs + [_full2d((DH, DH)), _full2d((1, DH))],
    out_specs=_node_out_specs, out_shape=_node_out_shape)


def _init_body(x_ref, w_ref, b_ref, w1a_ref, w1b_ref,
               out_ref, pa_out_ref, pb_out_ref):
    h0 = jnp.dot(x_ref[...], w_ref[...],
                 preferred_element_type=_f32) + b_ref[...]
    out_ref[...] = h0
    pa_out_ref[...] = jnp.dot(h0, w1a_ref[...], preferred_element_type=_f32)
    pb_out_ref[...] = jnp.dot(h0, w1b_ref[...], preferred_element_type=_f32)


_init_call = pl.pallas_call(
    _init_body,
    grid=(NACC // NB,),
    in_specs=[
        pl.BlockSpec((NB, DH), lambda i: (i, 0)),
        _full2d((DH, DH)), _full2d((1, DH)),
        _full2d((DH, DH)), _full2d((DH, DH)),
    ],
    out_specs=[pl.BlockSpec((NB, DH), lambda i: (i, 0))] * 3,
    out_shape=[jax.ShapeDtypeStruct((N, DH), _f32)] * 3,
)


def kernel(x, pos, edge_index, edge_attr, W_in, b_in, edge_W1, edge_b1,
           edge_W2, edge_b2, coord_W1, coord_b1, coord_W2, node_W1, node_b1,
           node_W2, node_b2, W_out, b_out):
    row = edge_index[0]
    col = edge_index[1]
    padn = EP - E
    rowg = jnp.concatenate([row, jnp.zeros((padn,), _i32)]).reshape(NW, CH, CB)
    colg = jnp.concatenate([col, jnp.zeros((padn,), _i32)]).reshape(NW, CH, CB)
    cols = jnp.concatenate([col, jnp.full((padn,), N, _i32)]).reshape(NW, CH, CB)
    ea_pad = jnp.concatenate([edge_attr, jnp.zeros((padn, DE), _f32)], axis=0)
    zeros_tile = jnp.zeros((NROWS_TILE, DH), _f32)
    zeros_flat = jnp.zeros((PFLAT,), _f32)
    posp = jnp.zeros((NACC, 3), _f32).at[:N].set(pos).reshape(NPACK, DH)

    gather_sc, scatter_sc = _build_sc_kernels()
    b1r = lambda b: b.reshape(1, DH)
    t, pa, pb = _init_call(x, W_in, b_in.reshape(1, DH),
                           edge_W1[0, :DH], edge_W1[0, DH:2 * DH])
    for l in range(NLAYERS):
        src, dst, aux = gather_sc(pb, pa, posp.reshape(PFLAT), rowg, colg)
        payf, payc = _edge_call(
            src, dst, aux, ea_pad,
            edge_W1[l, 2 * DH:2 * DH + 1], edge_W1[l, 2 * DH + 1:],
            b1r(edge_b1[l]),
            edge_W2[l], b1r(edge_b2[l]),
            coord_W1[l], b1r(coord_b1[l]),
            coord_W2[l].reshape(1, DH),
        )
        pf, pc = scatter_sc(payf, payc, cols, zeros_tile, zeros_flat)
        args = (t, posp, pf, pc.reshape(NW, NPACK, DH),
                node_W1[l, :DH], node_W1[l, DH:], b1r(node_b1[l]),
                node_W2[l], b1r(node_b2[l]))
        if l < NLAYERS - 1:
            t, posp, pa, pb = _node_call(
                *args, edge_W1[l + 1, :DH], edge_W1[l + 1, DH:2 * DH])
        else:
            out_x, posp = _node_final_call(*args, W_out, b_out.reshape(1, DH))
    pos_out = posp.reshape(NACC, 3)[:N]
    return (out_x, pos_out)


# speedup vs baseline: 1.3776x; 1.2279x over previous
import functools

import jax
import jax.numpy as jnp
from jax import lax
from jax.experimental import pallas as pl
from jax.experimental.pallas import tpu as pltpu
from jax.experimental.pallas import tpu_sc as plsc

N = 10000
E = 320000
DH = 128
DE = 16
NLAYERS = 4

NC, NS = 2, 16
NW = NC * NS
CB = 128
CH = 79
EP = NW * CH * CB
NCHUNK = NW * CH
NACC = 10240
PFLAT = NACC * 3
NPACK = PFLAT // DH
NROWS_TILE = NACC // NS

EB = 512
CPB = EB // CB
NB = 1024

_f32 = jnp.float32
_i32 = jnp.int32


def _silu(z):
    return z * jax.nn.sigmoid(z)


@functools.cache
def _build_sc_kernels():
    mesh = plsc.VectorSubcoreMesh(core_axis_name="c", subcore_axis_name="s")

    @functools.partial(
        pl.kernel,
        out_type=[
            jax.ShapeDtypeStruct((EP, DH), _f32),
            jax.ShapeDtypeStruct((EP, DH), _f32),
            jax.ShapeDtypeStruct((NCHUNK, 4, CB), _f32),
        ],
        mesh=mesh,
        scratch_types=[
            pltpu.VMEM((CH, CB), _i32),
            pltpu.VMEM((CH, CB), _i32),
            pltpu.VMEM((PFLAT,), _f32),
            pltpu.VMEM((CB, DH), _f32),
            pltpu.VMEM((CB, DH), _f32),
            pltpu.VMEM((4, CB), _f32),
            pltpu.SemaphoreType.DMA,
            pltpu.SemaphoreType.DMA,
        ],
        compiler_params=pltpu.CompilerParams(needs_layout_passes=False),
    )
    def gather_sc(pb_hbm, pa_hbm, posp_hbm, rowg_hbm, colg_hbm, src_out,
                  dst_out, aux_out, idxr_v, idxc_v, posp_v, bufr_v, bufc_v,
                  aux_v, semr, semc):
        wid = lax.axis_index("s") * NC + lax.axis_index("c")
        pltpu.sync_copy(rowg_hbm.at[wid], idxr_v)
        pltpu.sync_copy(colg_hbm.at[wid], idxc_v)
        pltpu.sync_copy(posp_hbm, posp_v)
        base = wid * (CH * CB)
        cbase = wid * CH

        def body(j, carry):
            off = base + j * CB
            cr = pltpu.async_copy(pb_hbm.at[idxr_v.at[j]], bufr_v, semr)
            cc = pltpu.async_copy(pa_hbm.at[idxc_v.at[j]], bufc_v, semc)
            for g in range(CB // 16):
                sl = pl.ds(g * 16, 16)
                ar = idxr_v[j, sl] * 3
                ac = idxc_v[j, sl] * 3
                prx = plsc.load_gather(posp_v, [ar])
                pry = plsc.load_gather(posp_v, [ar + 1])
                prz = plsc.load_gather(posp_v, [ar + 2])
                pcx = plsc.load_gather(posp_v, [ac])
                pcy = plsc.load_gather(posp_v, [ac + 1])
                pcz = plsc.load_gather(posp_v, [ac + 2])
                cdx = prx - pcx
                cdy = pry - pcy
                cdz = prz - pcz
                aux_v[0, sl] = cdx
                aux_v[1, sl] = cdy
                aux_v[2, sl] = cdz
                aux_v[3, sl] = cdx * cdx + cdy * cdy + cdz * cdz
            cr.wait()
            pltpu.sync_copy(bufr_v, src_out.at[pl.ds(off, CB)])
            cc.wait()
            pltpu.sync_copy(bufc_v, dst_out.at[pl.ds(off, CB)])
            pltpu.sync_copy(aux_v, aux_out.at[cbase + j])
            return carry

        lax.fori_loop(0, CH, body, 0)

    @functools.partial(
        pl.kernel,
        out_type=[
            jax.ShapeDtypeStruct((NC, NACC, DH), _f32),
            jax.ShapeDtypeStruct((NW, PFLAT), _f32),
        ],
        mesh=mesh,
        scratch_types=[
            pltpu.VMEM((1, CB), _i32),
            pltpu.VMEM((CB, DH), _f32),
            pltpu.VMEM((4, CB), _f32),
            pltpu.VMEM((PFLAT,), _f32),
            pltpu.VMEM_SHARED((NACC, DH), _f32),
        ],
        compiler_params=pltpu.CompilerParams(needs_layout_passes=False),
    )
    def scatter_sc(payf_hbm, payc_hbm, cols_hbm, zeros_hbm, zflat_hbm,
                   outf_hbm, outc_hbm, idx_v, bufe_v, bufc_v, accc_v, accf_sh):
        cid = lax.axis_index("c")
        sid = lax.axis_index("s")
        wid = sid * NC + cid
        tsl = pl.ds(sid * NROWS_TILE, NROWS_TILE)
        pltpu.sync_copy(zeros_hbm, accf_sh.at[tsl])
        pltpu.sync_copy(zflat_hbm, accc_v)
        plsc.subcore_barrier()
        base = wid * (CH * CB)

        def body(j, carry):
            off = base + j * CB
            pltpu.sync_copy(cols_hbm.at[wid, pl.ds(j, 1)], idx_v)
            pltpu.sync_copy(payf_hbm.at[pl.ds(off, CB)], bufe_v)
            pltpu.sync_copy(payc_hbm.at[wid * CH + j], bufc_v)
            pltpu.sync_copy(bufe_v, accf_sh.at[idx_v.at[0]], add=True)
            for g in range(CB // 16):
                sl = pl.ds(g * 16, 16)
                addr = idx_v[0, sl] * 3
                for c in range(3):
                    plsc.addupdate_scatter(accc_v, [addr + c], bufc_v[c, sl])
            return carry

        lax.fori_loop(0, CH, body, 0)
        pltpu.sync_copy(accc_v, outc_hbm.at[wid])
        plsc.subcore_barrier()
        pltpu.sync_copy(accf_sh.at[tsl], outf_hbm.at[cid, tsl])

    return gather_sc, scatter_sc


def _edge_body(src_ref, dst_ref, aux_ref, ea_ref, w1d, w1e, b1,
               w2, b2, cw1, cb1, cw2r, outf_ref, outc_ref):
    auxt = jnp.transpose(aux_ref[...].reshape(CPB * 4, CB))
    em = jnp.concatenate(
        [auxt[:, 4 * r:4 * r + 4] for r in range(CPB)], axis=0)
    cd = em[:, 0:3]
    dist = em[:, 3:4]
    z = (src_ref[...] + dst_ref[...]
         + dist * w1d[...]
         + jnp.dot(ea_ref[...], w1e[...], preferred_element_type=_f32)
         + b1[...])
    e1_ = _silu(z)
    z2 = jnp.dot(e1_, w2[...], preferred_element_type=_f32) + b2[...]
    ef = _silu(z2)
    z3 = jnp.dot(ef, cw1[...], preferred_element_type=_f32) + cb1[...]
    c1 = _silu(z3)
    s = jnp.sum(c1 * cw2r[...], axis=1, keepdims=True)
    outf_ref[...] = ef
    cu = cd * s
    cu4 = jnp.concatenate([cu, jnp.zeros((EB, 1), _f32)], axis=1)
    stacked = jnp.concatenate(
        [cu4[r * CB:(r + 1) * CB] for r in range(CPB)], axis=1)
    outc_ref[...] = jnp.transpose(stacked).reshape(CPB, 4, CB)


def _full2d(shape):
    return pl.BlockSpec(shape, lambda i: (0, 0))


_edge_call = pl.pallas_call(
    _edge_body,
    grid=(EP // EB,),
    in_specs=[
        pl.BlockSpec((EB, DH), lambda i: (i, 0)),
        pl.BlockSpec((EB, DH), lambda i: (i, 0)),
        pl.BlockSpec((CPB, 4, CB), lambda i: (i, 0, 0)),
        pl.BlockSpec((EB, DE), lambda i: (i, 0)),
        _full2d((1, DH)),
        _full2d((DE, DH)), _full2d((1, DH)),
        _full2d((DH, DH)), _full2d((1, DH)),
        _full2d((DH, DH)), _full2d((1, DH)), _full2d((1, DH)),
    ],
    out_specs=[
        pl.BlockSpec((EB, DH), lambda i: (i, 0)),
        pl.BlockSpec((CPB, 4, CB), lambda i: (i, 0, 0)),
    ],
    out_shape=[
        jax.ShapeDtypeStruct((EP, DH), _f32),
        jax.ShapeDtypeStruct((NCHUNK, 4, CB), _f32),
    ],
)


def _node_common(t_ref, pp_ref, pf_ref, pc_ref, nw1a, nw1b, nb1, nw2, nb2):
    h = t_ref[...]
    pf = pf_ref[...]
    aggf = pf[0] + pf[1]
    z = (jnp.dot(h, nw1a[...], preferred_element_type=_f32)
         + jnp.dot(aggf, nw1b[...], preferred_element_type=_f32)
         + nb1[...])
    u = _silu(z)
    h_new = jnp.dot(u, nw2[...], preferred_element_type=_f32) + nb2[...]
    pp_new = pp_ref[...] + jnp.sum(pc_ref[...], axis=0)
    return h_new, pp_new


def _node_body(t_ref, pp_ref, pf_ref, pc_ref, nw1a, nw1b, nb1, nw2, nb2,
               w1a_n, w1b_n, h_out_ref, pp_out_ref, pa_out_ref, pb_out_ref):
    h_new, pp_new = _node_common(t_ref, pp_ref, pf_ref, pc_ref,
                                 nw1a, nw1b, nb1, nw2, nb2)
    h_out_ref[...] = h_new
    pp_out_ref[...] = pp_new
    pa_out_ref[...] = jnp.dot(h_new, w1a_n[...], preferred_element_type=_f32,
                              precision=lax.Precision.HIGHEST)
    pb_out_ref[...] = jnp.dot(h_new, w1b_n[...], preferred_element_type=_f32,
                              precision=lax.Precision.HIGHEST)


def _node_final_body(t_ref, pp_ref, pf_ref, pc_ref, nw1a, nw1b, nb1, nw2, nb2,
                     wout, bout, x_out_ref, pp_out_ref):
    h_new, pp_new = _node_common(t_ref, pp_ref, pf_ref, pc_ref,
                                 nw1a, nw1b, nb1, nw2, nb2)
    x_out_ref[...] = (jnp.dot(h_new, wout[...],
                              preferred_element_type=_f32) + bout[...])
    pp_out_ref[...] = pp_new


_PPB = NB * 3 // DH

_node_in_specs = [
    pl.BlockSpec((NB, DH), lambda i: (i, 0)),
    pl.BlockSpec((_PPB, DH), lambda i: (i, 0)),
    pl.BlockSpec((NC, NB, DH), lambda i: (0, i, 0)),
    pl.BlockSpec((NW, _PPB, DH), lambda i: (0, i, 0)),
    _full2d((DH, DH)), _full2d((DH, DH)), _full2d((1, DH)),
    _full2d((DH, DH)), _full2d((1, DH)),
]
_node_out_specs = [
    pl.BlockSpec((NB, DH), lambda i: (i, 0)),
    pl.BlockSpec((_PPB, DH), lambda i: (i, 0)),
]
_node_out_shape = [
    jax.ShapeDtypeStruct((N, DH), _f32),
    jax.ShapeDtypeStruct((NPACK, DH), _f32),
]

_node_call = pl.pallas_call(
    _node_body, grid=(NACC // NB,),
    in_specs=_node_in_specs + [_full2d((DH, DH)), _full2d((DH, DH))],
    out_specs=_node_out_specs + [
        pl.BlockSpec((NB, DH), lambda i: (i, 0)),
        pl.BlockSpec((NB, DH), lambda i: (i, 0)),
    ],
    out_shape=_node_out_shape + [
        jax.ShapeDtypeStruct((N, DH), _f32),
        jax.ShapeDtypeStruct((N, DH), _f32),
    ])

_node_final_call = pl.pallas_call(
    _node_final_body, grid=(NACC // NB,),
    in_specs=_node_in_specs + [_full2d((DH, DH)), _full2d((1, DH))],
    out_specs=_node_out_specs, out_shape=_node_out_shape)


def _init_body(x_ref, w_ref, b_ref, w1a_ref, w1b_ref,
               out_ref, pa_out_ref, pb_out_ref):
    h0 = jnp.dot(x_ref[...], w_ref[...],
                 preferred_element_type=_f32) + b_ref[...]
    out_ref[...] = h0
    pa_out_ref[...] = jnp.dot(h0, w1a_ref[...], preferred_element_type=_f32,
                              precision=lax.Precision.HIGHEST)
    pb_out_ref[...] = jnp.dot(h0, w1b_ref[...], preferred_element_type=_f32,
                              precision=lax.Precision.HIGHEST)


_init_call = pl.pallas_call(
    _init_body,
    grid=(NACC // NB,),
    in_specs=[
        pl.BlockSpec((NB, DH), lambda i: (i, 0)),
        _full2d((DH, DH)), _full2d((1, DH)),
        _full2d((DH, DH)), _full2d((DH, DH)),
    ],
    out_specs=[pl.BlockSpec((NB, DH), lambda i: (i, 0))] * 3,
    out_shape=[jax.ShapeDtypeStruct((N, DH), _f32)] * 3,
)


def kernel(x, pos, edge_index, edge_attr, W_in, b_in, edge_W1, edge_b1,
           edge_W2, edge_b2, coord_W1, coord_b1, coord_W2, node_W1, node_b1,
           node_W2, node_b2, W_out, b_out):
    row = edge_index[0]
    col = edge_index[1]
    padn = EP - E
    rowg = jnp.concatenate([row, jnp.zeros((padn,), _i32)]).reshape(NW, CH, CB)
    colg = jnp.concatenate([col, jnp.zeros((padn,), _i32)]).reshape(NW, CH, CB)
    cols = jnp.concatenate([col, jnp.full((padn,), N, _i32)]).reshape(NW, CH, CB)
    ea_pad = jnp.concatenate([edge_attr, jnp.zeros((padn, DE), _f32)], axis=0)
    zeros_tile = jnp.zeros((NROWS_TILE, DH), _f32)
    zeros_flat = jnp.zeros((PFLAT,), _f32)
    posp = jnp.zeros((NACC, 3), _f32).at[:N].set(pos).reshape(NPACK, DH)

    gather_sc, scatter_sc = _build_sc_kernels()
    b1r = lambda b: b.reshape(1, DH)
    t, pa, pb = _init_call(x, W_in, b_in.reshape(1, DH),
                           edge_W1[0, :DH], edge_W1[0, DH:2 * DH])
    for l in range(NLAYERS):
        src, dst, aux = gather_sc(pb, pa, posp.reshape(PFLAT), rowg, colg)
        payf, payc = _edge_call(
            src, dst, aux, ea_pad,
            edge_W1[l, 2 * DH:2 * DH + 1], edge_W1[l, 2 * DH + 1:],
            b1r(edge_b1[l]),
            edge_W2[l], b1r(edge_b2[l]),
            coord_W1[l], b1r(coord_b1[l]),
            coord_W2[l].reshape(1, DH),
        )
        pf, pc = scatter_sc(payf, payc, cols, zeros_tile, zeros_flat)
        args = (t, posp, pf, pc.reshape(NW, NPACK, DH),
                node_W1[l, :DH], node_W1[l, DH:], b1r(node_b1[l]),
                node_W2[l], b1r(node_b2[l]))
        if l < NLAYERS - 1:
            t, posp, pa, pb = _node_call(
                *args, edge_W1[l + 1, :DH], edge_W1[l + 1, DH:2 * DH])
        else:
            out_x, posp = _node_final_call(*args, W_out, b_out.reshape(1, DH))
    pos_out = posp.reshape(NACC, 3)[:N]
    return (out_x, pos_out)
